# baseline (speedup 1.0000x reference)
import jax
import jax.numpy as jnp
from jax.experimental import pallas as pl
from jax.experimental.pallas import tpu as pltpu
from jax.experimental.pallas import tpu_sc as plsc

_WINDOW = 256
_LANES = 16


def kernel(timedelta, table):
    B, T = timedelta.shape
    V, D = table.shape
    N = B * T
    W = _WINDOW

    idx = timedelta.reshape(-1).astype(jnp.int32)
    pair_idx = (idx[0::2] * V + idx[1::2]).reshape(1, N // 2)
    pair_table = jnp.concatenate(
        [
            jnp.broadcast_to(table[:, None, :], (V, V, D)),
            jnp.broadcast_to(table[None, :, :], (V, V, D)),
        ],
        axis=-1,
    ).reshape(V * V, 2 * D)

    mesh = plsc.VectorSubcoreMesh(core_axis_name="core", subcore_axis_name="subcore")

    @pl.kernel(
        out_type=jax.ShapeDtypeStruct((N, D), table.dtype),
        mesh=mesh,
        scratch_types=[pltpu.VMEM((W // 2, 2 * D), table.dtype)],
    )
    def _lookup(table_hbm, i_hbm, o_hbm, scratch):
        def body(i_vmem, o_vmem):
            pltpu.sync_copy(table_hbm.at[i_vmem.at[0]], scratch)

            @pl.loop(0, W // 2)
            def _(j):
                for c in range(D // _LANES):
                    o_vmem[2 * j, pl.ds(c * _LANES, _LANES)] = scratch[
                        j, pl.ds(c * _LANES, _LANES)
                    ]
                for c in range(D // _LANES):
                    o_vmem[2 * j + 1, pl.ds(c * _LANES, _LANES)] = scratch[
                        j, pl.ds(D + c * _LANES, _LANES)
                    ]

        pltpu.emit_pipeline(
            body,
            grid=(N // W,),
            in_specs=[pl.BlockSpec((1, W // 2), index_map=lambda i: (0, i))],
            out_specs=[pl.BlockSpec((W, D), index_map=lambda i: (i, 0))],
            core_axis_name=("core", "subcore"),
            dimension_semantics=(pltpu.PARALLEL,),
        )(i_hbm, o_hbm)

    out = _lookup(pair_table, pair_idx)
    return out.reshape(B, T, D)
